# baseline (speedup 1.0000x reference)
import dataclasses
import functools

import jax
import jax.numpy as jnp
from jax import lax
from jax.experimental import pallas as pl
from jax.experimental.pallas import tpu as pltpu
from jax.experimental.pallas import tpu_sc as plsc

B, C, H, W = 16, 256, 32, 32
HW = H * W
K = 1024

_SC_INFO = plsc.get_sparse_core_info()
NC, NS, L = _SC_INFO.num_cores, _SC_INFO.num_subcores, _SC_INFO.num_lanes
NW = NC * NS
CPW = C // NW

NCHUNK = 2
CB = B // NCHUNK


def _match_kernel(x_ref, bank_ref, idx_ref, pn_ref):
    b = pl.program_id(0)

    @pl.when(b == 0)
    def _():
        bank = bank_ref[...]
        norm = jnp.sqrt(jnp.sum(bank * bank, axis=1, keepdims=True))
        pn_ref[...] = bank / jnp.maximum(norm, 1e-12)

    xb = x_ref[0]
    xnorm = jnp.sqrt(jnp.sum(xb * xb, axis=0, keepdims=True))
    xn = xb / jnp.maximum(xnorm, 1e-12)

    sim = jnp.dot(pn_ref[...], xn, preferred_element_type=jnp.float32)

    m = jnp.max(sim, axis=0, keepdims=True)
    iota_k = lax.broadcasted_iota(jnp.int32, (K, HW), 0)
    masked = jnp.where(sim == m, iota_k, K)
    idx_ref[0] = jnp.min(masked, axis=0, keepdims=True)


def _match(xf, bank):
    return pl.pallas_call(
        _match_kernel,
        grid=(CB,),
        in_specs=[
            pl.BlockSpec((1, C, HW), lambda b: (b, 0, 0)),
            pl.BlockSpec((K, C), lambda b: (0, 0)),
        ],
        out_specs=pl.BlockSpec((1, 1, HW), lambda b: (b, 0, 0)),
        out_shape=jax.ShapeDtypeStruct((CB, 1, HW), jnp.int32),
        scratch_shapes=[pltpu.VMEM((K, C), jnp.float32)],
    )(xf, bank)


_SC_PARAMS = pltpu.CompilerParams()
if "needs_layout_passes" in pltpu.CompilerParams.__dataclass_fields__:
    _SC_PARAMS = dataclasses.replace(_SC_PARAMS, needs_layout_passes=False)

_JSTEP = 4


@functools.partial(
    pl.kernel,
    mesh=plsc.VectorSubcoreMesh(core_axis_name="c", subcore_axis_name="s"),
    compiler_params=_SC_PARAMS,
    out_type=jax.ShapeDtypeStruct((CB, C, HW), jnp.float32),
    scratch_types=[
        pltpu.VMEM((CPW, HW), jnp.float32),
        pltpu.VMEM((CB, HW), jnp.int32),
        pltpu.VMEM((2, CPW, HW), jnp.float32),
        pltpu.SemaphoreType.DMA,
    ],
)
def _sc_gather(bank_t_hbm, idx_hbm, out_hbm, brows, idxv, ostage, sem):
    wid = lax.axis_index("s") * NC + lax.axis_index("c")
    cbase = wid * CPW
    pltpu.sync_copy(bank_t_hbm.at[pl.ds(cbase, CPW)], brows)
    pltpu.sync_copy(idx_hbm, idxv)

    def _dma(b):
        return pltpu.make_async_copy(
            ostage.at[b % 2], out_hbm.at[b, pl.ds(cbase, CPW)], sem)

    for b in range(CB):
        par = b % 2
        if b >= 2:
            _dma(b - 2).wait()

        @pl.loop(0, HW // L, step=_JSTEP)
        def _(j0):
            ivs = [idxv[b, pl.ds((j0 + u) * L, L)] for u in range(_JSTEP)]
            vals = [
                plsc.load_gather(brows, [jnp.full((L,), cl, jnp.int32), iv])
                for iv in ivs
                for cl in range(CPW)
            ]
            i = 0
            for u in range(_JSTEP):
                for cl in range(CPW):
                    ostage[par, cl, pl.ds((j0 + u) * L, L)] = vals[i]
                    i += 1

        _dma(b).start()

    _dma(CB - 2).wait()
    _dma(CB - 1).wait()


def kernel(x, prototype_bank):
    bank_t = prototype_bank.T
    outs, idxs = [], []
    for ci in range(NCHUNK):
        xc = x[ci * CB:(ci + 1) * CB].reshape(CB, C, HW)
        idx3 = _match(xc, prototype_bank)
        idxc = idx3.reshape(CB, HW)
        outs.append(_sc_gather(bank_t, idxc))
        idxs.append(idxc)
    out = jnp.concatenate(outs, axis=0)
    idx = jnp.concatenate(idxs, axis=0)
    return out.reshape(B, C, H, W), idx


# speedup vs baseline: 2.1703x; 2.1703x over previous
import dataclasses
import functools

import jax
import jax.numpy as jnp
from jax import lax
from jax.experimental import pallas as pl
from jax.experimental.pallas import tpu as pltpu
from jax.experimental.pallas import tpu_sc as plsc

B, C, H, W = 16, 256, 32, 32
HW = H * W
K = 1024

_SC_INFO = plsc.get_sparse_core_info()
NC, NS, L = _SC_INFO.num_cores, _SC_INFO.num_subcores, _SC_INFO.num_lanes
NW = NC * NS
PPW = B * HW // NW
GCH = 128
NCHUNKS = PPW // GCH


def _match_kernel(x_ref, bank_t_ref, idx_ref, pn_t_ref):
    b = pl.program_id(0)

    @pl.when(b == 0)
    def _():
        bank_t = bank_t_ref[...]
        norm = jnp.sqrt(jnp.sum(bank_t * bank_t, axis=0, keepdims=True))
        pn_t_ref[...] = bank_t / jnp.maximum(norm, 1e-12)

    xb = x_ref[0]
    xnorm = jnp.sqrt(jnp.sum(xb * xb, axis=1, keepdims=True))
    xn = xb / jnp.maximum(xnorm, 1e-12)

    sim = jnp.dot(xn, pn_t_ref[...], preferred_element_type=jnp.float32)

    m = jnp.max(sim, axis=1, keepdims=True)
    iota_k = lax.broadcasted_iota(jnp.int32, (HW, K), 1)
    masked = jnp.where(sim == m, iota_k, K)
    idx_col = jnp.min(masked, axis=1, keepdims=True)
    idx_ref[0] = idx_col.reshape(HW // 128, 128)


def _match(xr, bank_t):
    return pl.pallas_call(
        _match_kernel,
        grid=(B,),
        in_specs=[
            pl.BlockSpec((1, HW, C), lambda b: (b, 0, 0)),
            pl.BlockSpec((C, K), lambda b: (0, 0)),
        ],
        out_specs=pl.BlockSpec((1, HW // 128, 128), lambda b: (b, 0, 0)),
        out_shape=jax.ShapeDtypeStruct((B, HW // 128, 128), jnp.int32),
        scratch_shapes=[pltpu.VMEM((C, K), jnp.float32)],
    )(xr, bank_t)


_SC_PARAMS = pltpu.CompilerParams()
if "needs_layout_passes" in pltpu.CompilerParams.__dataclass_fields__:
    _SC_PARAMS = dataclasses.replace(_SC_PARAMS, needs_layout_passes=False)


@functools.partial(
    pl.kernel,
    mesh=plsc.VectorSubcoreMesh(core_axis_name="c", subcore_axis_name="s"),
    compiler_params=_SC_PARAMS,
    out_type=jax.ShapeDtypeStruct((B * HW, C), jnp.float32),
    scratch_types=[
        pltpu.VMEM((NCHUNKS, GCH), jnp.int32),
        pltpu.VMEM((2, GCH, C), jnp.float32),
        pltpu.SemaphoreType.DMA,
        pltpu.SemaphoreType.DMA,
    ],
)
def _sc_gather(bank_hbm, idx_hbm, out_hbm, idxv, rows, gsem, wsem):
    wid = lax.axis_index("s") * NC + lax.axis_index("c")
    base = wid * PPW
    pltpu.sync_copy(idx_hbm.at[pl.ds(wid * NCHUNKS, NCHUNKS)], idxv)

    def _g(c):
        return pltpu.make_async_copy(
            bank_hbm.at[idxv.at[c]], rows.at[c % 2], gsem)

    def _w(c):
        return pltpu.make_async_copy(
            rows.at[c % 2], out_hbm.at[pl.ds(base + c * GCH, GCH)], wsem)

    _g(0).start()
    for c in range(NCHUNKS):
        _g(c).wait()
        _w(c).start()
        if c >= 1:
            _w(c - 1).wait()
        if c + 1 < NCHUNKS:
            _g(c + 1).start()
    _w(NCHUNKS - 1).wait()


def kernel(x, prototype_bank):
    xr = x.transpose(0, 2, 3, 1).reshape(B, HW, C)
    bank_t = prototype_bank.T
    idx8 = _match(xr, bank_t)
    idx = idx8.reshape(B, HW)
    idx2 = idx.reshape(B * HW // GCH, GCH)
    rows = _sc_gather(prototype_bank, idx2)
    out = rows.reshape(B, H, W, C).transpose(0, 3, 1, 2)
    return out, idx


# speedup vs baseline: 2.4440x; 1.1261x over previous
import dataclasses
import functools

import jax
import jax.numpy as jnp
from jax import lax
from jax.experimental import pallas as pl
from jax.experimental.pallas import tpu as pltpu
from jax.experimental.pallas import tpu_sc as plsc

B, C, H, W = 16, 256, 32, 32
HW = H * W
K = 1024

_SC_INFO = plsc.get_sparse_core_info()
NC, NS, L = _SC_INFO.num_cores, _SC_INFO.num_subcores, _SC_INFO.num_lanes
NW = NC * NS
MB = 4
PPW = B * HW // NW
GCH = 128
NCHUNKS = PPW // GCH


def _match_kernel(x_ref, bank_t_ref, idx_ref, pn_t_ref):
    b = pl.program_id(0)

    @pl.when(b == 0)
    def _():
        bank_t = bank_t_ref[...]
        norm = jnp.sqrt(jnp.sum(bank_t * bank_t, axis=0, keepdims=True))
        pn_t_ref[...] = bank_t / jnp.maximum(norm, 1e-12)

    xb = x_ref[...].reshape(MB * HW, C)
    xnorm = jnp.sqrt(jnp.sum(xb * xb, axis=1, keepdims=True))
    xn = xb / jnp.maximum(xnorm, 1e-12)

    sim = jnp.dot(xn, pn_t_ref[...], preferred_element_type=jnp.float32)

    idx_col = jnp.argmax(sim, axis=1).astype(jnp.int32)
    idx_ref[...] = idx_col.reshape(MB, HW // 128, 128)


def _match(xr, bank_t):
    return pl.pallas_call(
        _match_kernel,
        grid=(B // MB,),
        in_specs=[
            pl.BlockSpec((MB, HW, C), lambda b: (b, 0, 0)),
            pl.BlockSpec((C, K), lambda b: (0, 0)),
        ],
        out_specs=pl.BlockSpec((MB, HW // 128, 128), lambda b: (b, 0, 0)),
        out_shape=jax.ShapeDtypeStruct((B, HW // 128, 128), jnp.int32),
        scratch_shapes=[pltpu.VMEM((C, K), jnp.float32)],
    )(xr, bank_t)


_SC_PARAMS = pltpu.CompilerParams()
if "needs_layout_passes" in pltpu.CompilerParams.__dataclass_fields__:
    _SC_PARAMS = dataclasses.replace(_SC_PARAMS, needs_layout_passes=False)


@functools.partial(
    pl.kernel,
    mesh=plsc.VectorSubcoreMesh(core_axis_name="c", subcore_axis_name="s"),
    compiler_params=_SC_PARAMS,
    out_type=jax.ShapeDtypeStruct((B * HW, C), jnp.float32),
    scratch_types=[
        pltpu.VMEM((NCHUNKS, GCH), jnp.int32),
        pltpu.VMEM((2, GCH, C), jnp.float32),
        pltpu.SemaphoreType.DMA,
        pltpu.SemaphoreType.DMA,
    ],
)
def _sc_gather(bank_hbm, idx_hbm, out_hbm, idxv, rows, gsem, wsem):
    wid = lax.axis_index("s") * NC + lax.axis_index("c")
    base = wid * PPW
    pltpu.sync_copy(idx_hbm.at[pl.ds(wid * NCHUNKS, NCHUNKS)], idxv)

    def _g(c):
        return pltpu.make_async_copy(
            bank_hbm.at[idxv.at[c]], rows.at[c % 2], gsem)

    def _w(c):
        return pltpu.make_async_copy(
            rows.at[c % 2], out_hbm.at[pl.ds(base + c * GCH, GCH)], wsem)

    _g(0).start()
    for c in range(NCHUNKS):
        _g(c).wait()
        _w(c).start()
        if c >= 1:
            _w(c - 1).wait()
        if c + 1 < NCHUNKS:
            _g(c + 1).start()
    _w(NCHUNKS - 1).wait()


def kernel(x, prototype_bank):
    xr = x.transpose(0, 2, 3, 1).reshape(B, HW, C)
    bank_t = prototype_bank.T
    idx8 = _match(xr, bank_t)
    idx = idx8.reshape(B, HW)
    idx2 = idx.reshape(B * HW // GCH, GCH)
    rows = _sc_gather(prototype_bank, idx2)
    out = rows.reshape(B, H, W, C).transpose(0, 3, 1, 2)
    return out, idx


# speedup vs baseline: 2.4651x; 1.0086x over previous
import dataclasses
import functools

import jax
import jax.numpy as jnp
from jax import lax
from jax.experimental import pallas as pl
from jax.experimental.pallas import tpu as pltpu
from jax.experimental.pallas import tpu_sc as plsc

B, C, H, W = 16, 256, 32, 32
HW = H * W
K = 1024

_SC_INFO = plsc.get_sparse_core_info()
NC, NS, L = _SC_INFO.num_cores, _SC_INFO.num_subcores, _SC_INFO.num_lanes
NW = NC * NS
MB = 4
PPW = B * HW // NW
GCH = 128
NCHUNKS = PPW // GCH


def _match_kernel(x_ref, bank_t_ref, idx_ref, pn_t_ref):
    b = pl.program_id(0)

    @pl.when(b == 0)
    def _():
        bank_t = bank_t_ref[...]
        norm = jnp.sqrt(jnp.sum(bank_t * bank_t, axis=0, keepdims=True))
        pn_t_ref[...] = bank_t / jnp.maximum(norm, 1e-12)

    xb = x_ref[...].reshape(MB * HW, C)
    xnorm = jnp.sqrt(jnp.sum(xb * xb, axis=1, keepdims=True))
    xn = xb / jnp.maximum(xnorm, 1e-12)

    sim = jnp.dot(xn, pn_t_ref[...], preferred_element_type=jnp.float32)

    idx_col = jnp.argmax(sim, axis=1).astype(jnp.int32)
    idx_ref[...] = idx_col.reshape(MB, HW // 128, 128)


def _match(xr, bank_t):
    return pl.pallas_call(
        _match_kernel,
        grid=(B // MB,),
        in_specs=[
            pl.BlockSpec((MB, HW, C), lambda b: (b, 0, 0)),
            pl.BlockSpec((C, K), lambda b: (0, 0)),
        ],
        out_specs=pl.BlockSpec((MB, HW // 128, 128), lambda b: (b, 0, 0)),
        out_shape=jax.ShapeDtypeStruct((B, HW // 128, 128), jnp.int32),
        scratch_shapes=[pltpu.VMEM((C, K), jnp.float32)],
    )(xr, bank_t)


_SC_PARAMS = pltpu.CompilerParams()
if "needs_layout_passes" in pltpu.CompilerParams.__dataclass_fields__:
    _SC_PARAMS = dataclasses.replace(_SC_PARAMS, needs_layout_passes=False)


@functools.partial(
    pl.kernel,
    mesh=plsc.VectorSubcoreMesh(core_axis_name="c", subcore_axis_name="s"),
    compiler_params=_SC_PARAMS,
    out_type=jax.ShapeDtypeStruct((B * HW, C), jnp.float32),
    scratch_types=[
        pltpu.VMEM((NCHUNKS, GCH), jnp.int32),
        pltpu.VMEM((3, GCH, C), jnp.float32),
        pltpu.SemaphoreType.DMA,
        pltpu.SemaphoreType.DMA,
    ],
)
def _sc_gather(bank_hbm, idx_hbm, out_hbm, idxv, rows, gsem, wsem):
    wid = lax.axis_index("s") * NC + lax.axis_index("c")
    base = wid * PPW
    pltpu.sync_copy(idx_hbm.at[pl.ds(wid * NCHUNKS, NCHUNKS)], idxv)

    def _g(c):
        return pltpu.make_async_copy(
            bank_hbm.at[idxv.at[c]], rows.at[c % 3], gsem)

    def _w(c):
        return pltpu.make_async_copy(
            rows.at[c % 3], out_hbm.at[pl.ds(base + c * GCH, GCH)], wsem)

    _g(0).start()
    _g(1).start()
    for c in range(NCHUNKS):
        _g(c).wait()
        _w(c).start()
        if c >= 1:
            _w(c - 1).wait()
        if c + 2 < NCHUNKS:
            _g(c + 2).start()
    _w(NCHUNKS - 1).wait()


def kernel(x, prototype_bank):
    xr = x.transpose(0, 2, 3, 1).reshape(B, HW, C)
    bank_t = prototype_bank.T
    idx8 = _match(xr, bank_t)
    idx = idx8.reshape(B, HW)
    idx2 = idx.reshape(B * HW // GCH, GCH)
    rows = _sc_gather(prototype_bank, idx2)
    out = rows.reshape(B, H, W, C).transpose(0, 3, 1, 2)
    return out, idx


# speedup vs baseline: 2.5766x; 1.0453x over previous
import dataclasses
import functools

import jax
import jax.numpy as jnp
from jax import lax
from jax.experimental import pallas as pl
from jax.experimental.pallas import tpu as pltpu
from jax.experimental.pallas import tpu_sc as plsc

B, C, H, W = 16, 256, 32, 32
HW = H * W
K = 1024

_SC_INFO = plsc.get_sparse_core_info()
NC, NS, L = _SC_INFO.num_cores, _SC_INFO.num_subcores, _SC_INFO.num_lanes
NW = NC * NS
MB = 4
PPW = B * HW // NW
GCH = 128
NCHUNKS = PPW // GCH


def _match_kernel(x_ref, bank_ref, idx_ref, pn_t_ref):
    b = pl.program_id(0)

    @pl.when(b == 0)
    def _():
        bank_t = lax.transpose(bank_ref[...], (1, 0))
        norm = jnp.sqrt(jnp.sum(bank_t * bank_t, axis=0, keepdims=True))
        pn_t_ref[...] = bank_t / jnp.maximum(norm, 1e-12)

    xb = x_ref[...].reshape(MB * HW, C)
    xnorm = jnp.sqrt(jnp.sum(xb * xb, axis=1, keepdims=True))
    xn = xb / jnp.maximum(xnorm, 1e-12)

    sim = jnp.dot(xn, pn_t_ref[...], preferred_element_type=jnp.float32)

    idx_col = jnp.argmax(sim, axis=1).astype(jnp.int32)
    idx_ref[...] = idx_col.reshape(MB, HW // 128, 128)


def _match(xr, bank):
    return pl.pallas_call(
        _match_kernel,
        grid=(B // MB,),
        in_specs=[
            pl.BlockSpec((MB, HW, C), lambda b: (b, 0, 0)),
            pl.BlockSpec((K, C), lambda b: (0, 0)),
        ],
        out_specs=pl.BlockSpec((MB, HW // 128, 128), lambda b: (b, 0, 0)),
        out_shape=jax.ShapeDtypeStruct((B, HW // 128, 128), jnp.int32),
        scratch_shapes=[pltpu.VMEM((C, K), jnp.float32)],
    )(xr, bank)


_SC_PARAMS = pltpu.CompilerParams()
if "needs_layout_passes" in pltpu.CompilerParams.__dataclass_fields__:
    _SC_PARAMS = dataclasses.replace(_SC_PARAMS, needs_layout_passes=False)


@functools.partial(
    pl.kernel,
    mesh=plsc.VectorSubcoreMesh(core_axis_name="c", subcore_axis_name="s"),
    compiler_params=_SC_PARAMS,
    out_type=jax.ShapeDtypeStruct((B * HW, C), jnp.float32),
    scratch_types=[
        pltpu.VMEM((NCHUNKS, GCH), jnp.int32),
        pltpu.VMEM((3, GCH, C), jnp.float32),
        pltpu.SemaphoreType.DMA,
        pltpu.SemaphoreType.DMA,
    ],
)
def _sc_gather(bank_hbm, idx_hbm, out_hbm, idxv, rows, gsem, wsem):
    wid = lax.axis_index("s") * NC + lax.axis_index("c")
    base = wid * PPW
    pltpu.sync_copy(idx_hbm.at[pl.ds(wid * NCHUNKS, NCHUNKS)], idxv)

    def _g(c):
        return pltpu.make_async_copy(
            bank_hbm.at[idxv.at[c]], rows.at[c % 3], gsem)

    def _w(c):
        return pltpu.make_async_copy(
            rows.at[c % 3], out_hbm.at[pl.ds(base + c * GCH, GCH)], wsem)

    _g(0).start()
    _g(1).start()
    for c in range(NCHUNKS):
        _g(c).wait()
        _w(c).start()
        if c >= 1:
            _w(c - 1).wait()
        if c + 2 < NCHUNKS:
            _g(c + 2).start()
    _w(NCHUNKS - 1).wait()


def kernel(x, prototype_bank):
    xr = x.transpose(0, 2, 3, 1).reshape(B, HW, C)
    idx8 = _match(xr, prototype_bank)
    idx = idx8.reshape(B, HW)
    idx2 = idx.reshape(B * HW // GCH, GCH)
    rows = _sc_gather(prototype_bank, idx2)
    out = rows.reshape(B, H, W, C).transpose(0, 3, 1, 2)
    return out, idx
